# Initial kernel scaffold; baseline (speedup 1.0000x reference)
import jax
import jax.numpy as jnp
from jax.experimental import pallas as pl


def kernel(pillar_features, voxel_coords, observations, conv_w, bn_gamma, bn_beta):
    raise NotImplementedError("write your pallas kernel here")


# speedup vs baseline: 1.3103x; 1.3103x over previous
import functools

import jax
import jax.numpy as jnp
from jax import lax
from jax.experimental import pallas as pl
from jax.experimental.pallas import tpu as pltpu
from jax.experimental.pallas import tpu_sc as plsc

NX = 512
NY = 512
C = 64
B = 2
P = 32000
S = NX * NY
OUTC = 2 * C
FLAT_OUT = B * OUTC * S

NC = 2
NS = 16
L = 16
NW = NC * NS
TS = (B * S) // NW
NPV = P // L
NOV = TS // L


def _idx_body(coords_ref, flat_ref):
    c = coords_ref[...]
    flat_ref[...] = (c[0:1] * S + c[1:2] + c[2:3] * NX + c[3:4])


def _flat_indices(coords_t):
    return pl.pallas_call(
        _idx_body,
        out_shape=jax.ShapeDtypeStruct((1, P), jnp.int32),
    )(coords_t)


RS = 64
NBS = NY // RS


def _stats_body(obsp_ref, g_ref, xs_ref):
    i = pl.program_id(1)
    y0 = i * RS
    rows = obsp_ref[0, pl.ds(y0, RS + 8), :]
    cols = []
    for di in range(3):
        for dj in range(3):
            sl = lax.slice(rows, (di, dj), (di + RS, dj + NX))
            cols.append(jnp.reshape(sl, (1, RS * NX)))
    x2 = jnp.concatenate(cols, axis=0)
    g = lax.dot_general(x2, x2, (((1,), (1,)), ((), ())),
                        preferred_element_type=jnp.float32)
    xs = jnp.sum(x2, axis=1, keepdims=True)

    @pl.when(jnp.logical_and(pl.program_id(0) == 0, i == 0))
    def _():
        g_ref[...] = jnp.zeros_like(g_ref)
        xs_ref[...] = jnp.zeros_like(xs_ref)

    g_ref[...] += g
    xs_ref[...] += xs


def _stats(obsp):
    return pl.pallas_call(
        _stats_body,
        grid=(B, NBS),
        in_specs=[pl.BlockSpec((1, NY + 8, NX + 2), lambda b, i: (b, 0, 0))],
        out_specs=[
            pl.BlockSpec((9, 9), lambda b, i: (0, 0)),
            pl.BlockSpec((9, 1), lambda b, i: (0, 0)),
        ],
        out_shape=[
            jax.ShapeDtypeStruct((9, 9), jnp.float32),
            jax.ShapeDtypeStruct((9, 1), jnp.float32),
        ],
    )(obsp)


RW = 16
NBW = NY // RW


def _write_body(obsp_ref, w9_ref, g_ref, xs_ref, gam_ref, bet_ref, out_ref):
    i = pl.program_id(1)
    y0 = i * RW

    w9 = w9_ref[...]
    n = jnp.float32(B * S)
    mean = lax.dot_general(w9, xs_ref[...], (((1,), (0,)), ((), ())),
                           preferred_element_type=jnp.float32) / n
    wg = lax.dot_general(w9, g_ref[...], (((1,), (0,)), ((), ())),
                         preferred_element_type=jnp.float32)
    ex2 = jnp.sum(wg * w9, axis=1, keepdims=True) / n
    var = ex2 - mean * mean
    inv = lax.rsqrt(var + 1e-3)
    scale = gam_ref[...][:, None] * inv
    shift = bet_ref[...][:, None] - mean * scale

    rows = obsp_ref[0, pl.ds(y0, RW + 8), :]
    cols = []
    for di in range(3):
        for dj in range(3):
            sl = lax.slice(rows, (di, dj), (di + RW, dj + NX))
            cols.append(jnp.reshape(sl, (1, RW * NX)))
    x2 = jnp.concatenate(cols, axis=0)
    conv = lax.dot_general(w9, x2, (((1,), (0,)), ((), ())),
                           preferred_element_type=jnp.float32)
    feat = jnp.maximum(conv * scale + shift, 0.0)
    out_ref[0, :C] = jnp.zeros((C, RW, NX), jnp.float32)
    out_ref[0, C:] = jnp.reshape(feat, (C, RW, NX))


def _write(obsp, w9, g, xs, gamma, beta):
    return pl.pallas_call(
        _write_body,
        grid=(B, NBW),
        in_specs=[
            pl.BlockSpec((1, NY + 8, NX + 2), lambda b, i: (b, 0, 0)),
            pl.BlockSpec((C, 9), lambda b, i: (0, 0)),
            pl.BlockSpec((9, 9), lambda b, i: (0, 0)),
            pl.BlockSpec((9, 1), lambda b, i: (0, 0)),
            pl.BlockSpec((C,), lambda b, i: (0,)),
            pl.BlockSpec((C,), lambda b, i: (0,)),
        ],
        out_specs=pl.BlockSpec((1, OUTC, RW, NX), lambda b, i: (b, 0, i, 0)),
        out_shape=jax.ShapeDtypeStruct((B, OUTC, NY, NX), jnp.float32),
    )(obsp, w9, g, xs, gamma, beta)


def _sc_body(canvas, flat_hbm, feat_hbm,
             idx_v, owner_v, pos_v, pid_v, rows_v, oidx_v, vals_v,
             gsem, ssem):
    wid = lax.axis_index("s") * NC + lax.axis_index("c")
    base = wid * TS
    iota = lax.iota(jnp.int32, L)

    pltpu.sync_copy(flat_hbm, idx_v)

    def init_body(i, carry):
        owner_v[pl.ds(i * L, L)] = jnp.full((L,), -1, jnp.int32)
        return carry
    lax.fori_loop(0, NOV, init_body, 0)

    def own_body(i, carry):
        v = idx_v[pl.ds(i * L, L)]
        loc = v - base
        msk = jnp.logical_and(loc >= 0, loc < TS)
        locc = jnp.clip(loc, 0, TS - 1)
        plsc.store_scatter(owner_v, [locc], iota + i * L, mask=msk)
        return carry
    lax.fori_loop(0, NPV, own_body, 0)

    def cmp_body(i, n):
        o = owner_v[pl.ds(i * L, L)]
        msk = o >= 0
        plsc.store_compressed(pos_v.at[pl.ds(n, L)], iota + i * L, mask=msk)
        plsc.store_compressed(pid_v.at[pl.ds(n, L)], o, mask=msk)
        cnt = lax.reduce_max(plsc.all_reduce_population_count(msk), (0,))
        return n + cnt
    n = lax.fori_loop(0, NOV, cmp_body, 0)

    rem = lax.rem(n, L)

    @pl.when(rem > 0)
    def _():
        nb = n - rem
        pv = pos_v[pl.ds(nb, L)]
        dv = pid_v[pl.ds(nb, L)]
        neg = jnp.full((L,), jnp.iinfo(jnp.int32).min, jnp.int32)
        p0 = lax.reduce_max(jnp.where(iota == 0, pv, neg), (0,))
        d0 = lax.reduce_max(jnp.where(iota == 0, dv, neg), (0,))
        pos_v[pl.ds(nb, L)] = jnp.where(iota < rem, pv, p0)
        pid_v[pl.ds(nb, L)] = jnp.where(iota < rem, dv, d0)

    nv = lax.select(rem > 0, n // L + 1, n // L)

    def win_body(v, carry):
        pidv = pid_v[pl.ds(v * L, L)]
        posv = pos_v[pl.ds(v * L, L)]
        flatg = posv + base
        bidx = lax.shift_right_logical(flatg, 18)
        sidx = jnp.bitwise_and(flatg, S - 1)
        obase = lax.shift_left(bidx, 25) + sidx

        cp = pltpu.async_copy(feat_hbm.at[pidv], rows_v, gsem)
        cp.wait()

        def ch_body(j, c2):
            def sub_body(k, c3):
                c = j * 8 + k
                col = plsc.load_gather(
                    rows_v, [iota, jnp.full((L,), c, jnp.int32)])
                oidx_v[j, pl.ds(k * L, L)] = obase + c * S
                vals_v[j, pl.ds(k * L, L)] = col
                return c3
            lax.fori_loop(0, 8, sub_body, 0)
            return c2
        lax.fori_loop(0, 8, ch_body, 0)

        def dma_body(j, c2):
            cpx = pltpu.async_copy(vals_v.at[j], canvas.at[oidx_v.at[j]], ssem)
            cpx.wait()
            return c2
        lax.fori_loop(0, 8, dma_body, 0)
        return carry
    lax.fori_loop(0, nv, win_body, 0)


def _sc_scatter(canvas_ref, flat, feats):
    mesh = plsc.VectorSubcoreMesh(core_axis_name="c", subcore_axis_name="s",
                                  num_cores=NC, num_subcores=NS)
    k = pl.kernel(
        _sc_body,
        out_type=(),
        mesh=mesh,
        compiler_params=pltpu.CompilerParams(needs_layout_passes=False,
                                             use_tc_tiling_on_sc=False),
        scratch_types=[
            pltpu.VMEM((P,), jnp.int32),
            pltpu.VMEM((TS,), jnp.int32),
            pltpu.VMEM((TS,), jnp.int32),
            pltpu.VMEM((TS,), jnp.int32),
            pltpu.VMEM((L, C), jnp.float32),
            pltpu.VMEM((8, 8 * L), jnp.int32),
            pltpu.VMEM((8, 8 * L), jnp.float32),
            pltpu.SemaphoreType.DMA,
            pltpu.SemaphoreType.DMA,
        ],
    )
    return k(canvas_ref, flat, feats)


def kernel(pillar_features, voxel_coords, observations, conv_w, bn_gamma,
           bn_beta):
    obsp = jnp.pad(observations.reshape(B, NY, NX),
                   ((0, 0), (1, 7), (1, 1)))
    w9 = conv_w.reshape(C, 9)

    flat = _flat_indices(voxel_coords.T).reshape(P)
    g, xs = _stats(obsp)
    out = _write(obsp, w9, g, xs, bn_gamma, bn_beta)

    ref = jax.new_ref(jnp.reshape(out, (FLAT_OUT,)))
    _sc_scatter(ref, flat, pillar_features)
    return ref[...].reshape(B, OUTC, NY, NX)
